# Initial kernel scaffold; baseline (speedup 1.0000x reference)
import jax
import jax.numpy as jnp
from jax.experimental import pallas as pl


def kernel(edge_index, node_features, W1, att_src1, att_dst1, b1, W2, att_src2, att_dst2, b2):
    raise NotImplementedError("write your pallas kernel here")


# speedup vs baseline: 14.7716x; 14.7716x over previous
import functools

import jax
import jax.numpy as jnp
from jax import lax
from jax.experimental import pallas as pl
from jax.experimental.pallas import tpu as pltpu
from jax.experimental.pallas import tpu_sc as plsc

N = 10000
E = 320000
D = 128
NPAD = 10240
EPAD = 327680
NC = 2
NS = 16
NW = NC * NS
EPW = EPAD // NW
K = 128
NCHUNK = EPW // K
RPT = NPAD // NS
ZR = 64
BR = 512


def _proj_body(x_ref, w_ref, as_ref, ad_ref, h_ref, ao_ref, bo_ref):
    x = x_ref[...]
    h = lax.dot_general(x, w_ref[...], (((1,), (1,)), ((), ())),
                        preferred_element_type=jnp.float32)
    h_ref[...] = h
    ao_ref[...] = lax.dot_general(h, as_ref[...], (((1,), (0,)), ((), ())),
                                  preferred_element_type=jnp.float32)
    bo_ref[...] = lax.dot_general(h, ad_ref[...], (((1,), (0,)), ((), ())),
                                  preferred_element_type=jnp.float32)


def _project1(x, w, att_s, att_d):
    grid = NPAD // BR
    return pl.pallas_call(
        _proj_body,
        grid=(grid,),
        in_specs=[
            pl.BlockSpec((BR, D), lambda i: (i, 0)),
            pl.BlockSpec((D, D), lambda i: (0, 0)),
            pl.BlockSpec((D,), lambda i: (0,)),
            pl.BlockSpec((D,), lambda i: (0,)),
        ],
        out_specs=[
            pl.BlockSpec((BR, D), lambda i: (i, 0)),
            pl.BlockSpec((BR,), lambda i: (i,)),
            pl.BlockSpec((BR,), lambda i: (i,)),
        ],
        out_shape=[
            jax.ShapeDtypeStruct((NPAD, D), jnp.float32),
            jax.ShapeDtypeStruct((NPAD,), jnp.float32),
            jax.ShapeDtypeStruct((NPAD,), jnp.float32),
        ],
    )(x, w, att_s, att_d)


def _combine_proj_body(p_ref, d_ref, b_ref, w_ref, as_ref, ad_ref,
                       h_ref, ao_ref, bo_ref):
    ps = p_ref[0] + p_ref[1]
    den = jnp.maximum(d_ref[0] + d_ref[1], 1e-16)
    x = jnp.maximum(ps / den[:, None] + b_ref[...][None, :], 0.0)
    h = lax.dot_general(x, w_ref[...], (((1,), (1,)), ((), ())),
                        preferred_element_type=jnp.float32)
    h_ref[...] = h
    ao_ref[...] = lax.dot_general(h, as_ref[...], (((1,), (0,)), ((), ())),
                                  preferred_element_type=jnp.float32)
    bo_ref[...] = lax.dot_general(h, ad_ref[...], (((1,), (0,)), ((), ())),
                                  preferred_element_type=jnp.float32)


def _project2(part, den, b, w, att_s, att_d):
    grid = NPAD // BR
    return pl.pallas_call(
        _combine_proj_body,
        grid=(grid,),
        in_specs=[
            pl.BlockSpec((NC, BR, D), lambda i: (0, i, 0)),
            pl.BlockSpec((NC, BR), lambda i: (0, i)),
            pl.BlockSpec((D,), lambda i: (0,)),
            pl.BlockSpec((D, D), lambda i: (0, 0)),
            pl.BlockSpec((D,), lambda i: (0,)),
            pl.BlockSpec((D,), lambda i: (0,)),
        ],
        out_specs=[
            pl.BlockSpec((BR, D), lambda i: (i, 0)),
            pl.BlockSpec((BR,), lambda i: (i,)),
            pl.BlockSpec((BR,), lambda i: (i,)),
        ],
        out_shape=[
            jax.ShapeDtypeStruct((NPAD, D), jnp.float32),
            jax.ShapeDtypeStruct((NPAD,), jnp.float32),
            jax.ShapeDtypeStruct((NPAD,), jnp.float32),
        ],
    )(part, den, b, w, att_s, att_d)


def _final_body(p_ref, d_ref, b_ref, o_ref):
    ps = p_ref[0] + p_ref[1]
    den = jnp.maximum(d_ref[0] + d_ref[1], 1e-16)
    o_ref[...] = ps / den[:, None] + b_ref[...][None, :]


def _finalize(part, den, b):
    grid = NPAD // BR
    return pl.pallas_call(
        _final_body,
        grid=(grid,),
        in_specs=[
            pl.BlockSpec((NC, BR, D), lambda i: (0, i, 0)),
            pl.BlockSpec((NC, BR), lambda i: (0, i)),
            pl.BlockSpec((D,), lambda i: (0,)),
        ],
        out_specs=pl.BlockSpec((BR, D), lambda i: (i, 0)),
        out_shape=jax.ShapeDtypeStruct((NPAD, D), jnp.float32),
    )(part, den, b)


def _edge_body(src_hbm, dst_hbm, asrc_hbm, adst_hbm, h_hbm,
               part_hbm, den_hbm,
               asrc_l, adst_l, srcb, dstb, wb, rows, zrow, zden,
               out_acc, den_acc, sem):
    cid = lax.axis_index("c")
    sid = lax.axis_index("s")
    wid = cid * NS + sid

    pltpu.sync_copy(asrc_hbm, asrc_l)
    pltpu.sync_copy(adst_hbm, adst_l)

    zero16 = jnp.zeros((16,), jnp.float32)

    def _zrow_body(i, _):
        for j in range(8):
            zrow[i, pl.ds(j * 16, 16)] = zero16
        return 0
    lax.fori_loop(0, ZR, _zrow_body, 0)

    def _zden_body(i, _):
        zden[pl.ds(i * 16, 16)] = zero16
        return 0
    lax.fori_loop(0, RPT // 16, _zden_body, 0)

    r0 = sid * RPT
    for t in range(RPT // ZR):
        pltpu.sync_copy(zrow, out_acc.at[pl.ds(r0 + t * ZR, ZR), :])
    pltpu.sync_copy(zden, den_acc.at[pl.ds(r0, RPT)])

    def _mx_body(i, m):
        ms = jnp.maximum(m[0], asrc_l[pl.ds(i * 16, 16)])
        md = jnp.maximum(m[1], adst_l[pl.ds(i * 16, 16)])
        return ms, md
    neg = jnp.full((16,), -3.0e38, jnp.float32)
    mx_s, mx_d = lax.fori_loop(0, NPAD // 16, _mx_body, (neg, neg))
    hs = mx_s[0]
    hd = mx_d[0]
    for l in range(1, 16):
        hs = jnp.maximum(hs, mx_s[l])
        hd = jnp.maximum(hd, mx_d[l])
    gmax = hs + hd
    c = jnp.where(gmax > 0, gmax, 0.2 * gmax)

    plsc.subcore_barrier()

    ebase = wid * EPW

    def _chunk_body(j, _):
        base = ebase + j * K
        pltpu.sync_copy(src_hbm.at[pl.ds(base, K)], srcb)
        pltpu.sync_copy(dst_hbm.at[pl.ds(base, K)], dstb)

        def _grp_body(g, _):
            si = srcb[pl.ds(g * 16, 16)]
            di = dstb[pl.ds(g * 16, 16)]
            av = plsc.load_gather(asrc_l, [si])
            bv = plsc.load_gather(adst_l, [di])
            gv = av + bv
            ev = jnp.where(gv > 0, gv, 0.2 * gv)
            eid = base + g * 16 + lax.iota(jnp.int32, 16)
            wv = jnp.where(eid < E, jnp.exp(ev - c), 0.0)
            wb[pl.ds(g * 16, 16)] = wv
            return 0
        lax.fori_loop(0, K // 16, _grp_body, 0)

        pltpu.async_copy(h_hbm.at[srcb], rows, sem).wait()

        def _scale_body(g, _):
            wv = wb[pl.ds(g * 16, 16)]
            for l in range(16):
                wvec = jnp.full((16,), wv[l], jnp.float32)
                r = g * 16 + l
                for j2 in range(8):
                    sl = pl.ds(j2 * 16, 16)
                    rows[r, sl] = rows[r, sl] * wvec
            return 0
        lax.fori_loop(0, K // 16, _scale_body, 0)

        pltpu.sync_copy(rows, out_acc.at[dstb], add=True)
        pltpu.sync_copy(wb, den_acc.at[dstb], add=True)
        return 0
    lax.fori_loop(0, NCHUNK, _chunk_body, 0)

    plsc.subcore_barrier()

    pltpu.sync_copy(out_acc.at[pl.ds(r0, RPT), :],
                    part_hbm.at[cid, pl.ds(r0, RPT), :])
    pltpu.sync_copy(den_acc.at[pl.ds(r0, RPT)],
                    den_hbm.at[cid, pl.ds(r0, RPT)])


_edge_pass = pl.kernel(
    _edge_body,
    out_type=(
        jax.ShapeDtypeStruct((NC, NPAD, D), jnp.float32),
        jax.ShapeDtypeStruct((NC, NPAD), jnp.float32),
    ),
    mesh=plsc.VectorSubcoreMesh(core_axis_name="c", subcore_axis_name="s",
                                num_cores=NC, num_subcores=NS),
    compiler_params=pltpu.CompilerParams(needs_layout_passes=False),
    scratch_types=[
        pltpu.VMEM((NPAD,), jnp.float32),
        pltpu.VMEM((NPAD,), jnp.float32),
        pltpu.VMEM((K,), jnp.int32),
        pltpu.VMEM((K,), jnp.int32),
        pltpu.VMEM((K,), jnp.float32),
        pltpu.VMEM((K, D), jnp.float32),
        pltpu.VMEM((ZR, D), jnp.float32),
        pltpu.VMEM((RPT,), jnp.float32),
        pltpu.VMEM_SHARED((NPAD, D), jnp.float32),
        pltpu.VMEM_SHARED((NPAD,), jnp.float32),
        pltpu.SemaphoreType.DMA,
    ],
)


@jax.jit
def kernel(edge_index, node_features, W1, att_src1, att_dst1, b1,
           W2, att_src2, att_dst2, b2):
    src = jnp.pad(edge_index[0], (0, EPAD - E)).astype(jnp.int32)
    dst = jnp.pad(edge_index[1], (0, EPAD - E)).astype(jnp.int32)
    x = jnp.pad(node_features, ((0, NPAD - N), (0, 0)))

    h1, as1, ad1 = _project1(x, W1, att_src1, att_dst1)
    p1, d1 = _edge_pass(src, dst, as1, ad1, h1)
    h2, as2, ad2 = _project2(p1, d1, b1, W2, att_src2, att_dst2)
    p2, d2 = _edge_pass(src, dst, as2, ad2, h2)
    out = _finalize(p2, d2, b2)
    return out[:N]


# speedup vs baseline: 28.3630x; 1.9201x over previous
import functools

import jax
import jax.numpy as jnp
from jax import lax
from jax.experimental import pallas as pl
from jax.experimental.pallas import tpu as pltpu
from jax.experimental.pallas import tpu_sc as plsc

N = 10000
E = 320000
D = 128
NPAD = 10240
EPAD = 327680
NC = 2
NS = 16
NW = NC * NS
EPW = EPAD // NW
K = 128
NCHUNK = EPW // K
RPT = NPAD // NS
ZR = 64
BR = 512


def _proj_body(x_ref, w_ref, as_ref, ad_ref, h_ref, ao_ref, bo_ref):
    x = x_ref[...]
    h = lax.dot_general(x, w_ref[...], (((1,), (1,)), ((), ())),
                        preferred_element_type=jnp.float32)
    h_ref[...] = h
    ao_ref[...] = lax.dot_general(h, as_ref[...], (((1,), (0,)), ((), ())),
                                  preferred_element_type=jnp.float32)
    bo_ref[...] = lax.dot_general(h, ad_ref[...], (((1,), (0,)), ((), ())),
                                  preferred_element_type=jnp.float32)


def _project1(x, w, att_s, att_d):
    grid = NPAD // BR
    return pl.pallas_call(
        _proj_body,
        grid=(grid,),
        in_specs=[
            pl.BlockSpec((BR, D), lambda i: (i, 0)),
            pl.BlockSpec((D, D), lambda i: (0, 0)),
            pl.BlockSpec((D,), lambda i: (0,)),
            pl.BlockSpec((D,), lambda i: (0,)),
        ],
        out_specs=[
            pl.BlockSpec((BR, D), lambda i: (i, 0)),
            pl.BlockSpec((BR,), lambda i: (i,)),
            pl.BlockSpec((BR,), lambda i: (i,)),
        ],
        out_shape=[
            jax.ShapeDtypeStruct((NPAD, D), jnp.float32),
            jax.ShapeDtypeStruct((NPAD,), jnp.float32),
            jax.ShapeDtypeStruct((NPAD,), jnp.float32),
        ],
    )(x, w, att_s, att_d)


def _combine_proj_body(p_ref, d_ref, b_ref, w_ref, as_ref, ad_ref,
                       h_ref, ao_ref, bo_ref):
    ps = p_ref[0] + p_ref[1]
    den = jnp.maximum(d_ref[0] + d_ref[1], 1e-16)
    x = jnp.maximum(ps / den[:, None] + b_ref[...][None, :], 0.0)
    h = lax.dot_general(x, w_ref[...], (((1,), (1,)), ((), ())),
                        preferred_element_type=jnp.float32)
    h_ref[...] = h
    ao_ref[...] = lax.dot_general(h, as_ref[...], (((1,), (0,)), ((), ())),
                                  preferred_element_type=jnp.float32)
    bo_ref[...] = lax.dot_general(h, ad_ref[...], (((1,), (0,)), ((), ())),
                                  preferred_element_type=jnp.float32)


def _project2(part, den, b, w, att_s, att_d):
    grid = NPAD // BR
    return pl.pallas_call(
        _combine_proj_body,
        grid=(grid,),
        in_specs=[
            pl.BlockSpec((NC, BR, D), lambda i: (0, i, 0)),
            pl.BlockSpec((NC, BR), lambda i: (0, i)),
            pl.BlockSpec((D,), lambda i: (0,)),
            pl.BlockSpec((D, D), lambda i: (0, 0)),
            pl.BlockSpec((D,), lambda i: (0,)),
            pl.BlockSpec((D,), lambda i: (0,)),
        ],
        out_specs=[
            pl.BlockSpec((BR, D), lambda i: (i, 0)),
            pl.BlockSpec((BR,), lambda i: (i,)),
            pl.BlockSpec((BR,), lambda i: (i,)),
        ],
        out_shape=[
            jax.ShapeDtypeStruct((NPAD, D), jnp.float32),
            jax.ShapeDtypeStruct((NPAD,), jnp.float32),
            jax.ShapeDtypeStruct((NPAD,), jnp.float32),
        ],
    )(part, den, b, w, att_s, att_d)


def _final_body(p_ref, d_ref, b_ref, o_ref):
    ps = p_ref[0] + p_ref[1]
    den = jnp.maximum(d_ref[0] + d_ref[1], 1e-16)
    o_ref[...] = ps / den[:, None] + b_ref[...][None, :]


def _finalize(part, den, b):
    grid = NPAD // BR
    return pl.pallas_call(
        _final_body,
        grid=(grid,),
        in_specs=[
            pl.BlockSpec((NC, BR, D), lambda i: (0, i, 0)),
            pl.BlockSpec((NC, BR), lambda i: (0, i)),
            pl.BlockSpec((D,), lambda i: (0,)),
        ],
        out_specs=pl.BlockSpec((BR, D), lambda i: (i, 0)),
        out_shape=jax.ShapeDtypeStruct((NPAD, D), jnp.float32),
    )(part, den, b)


def _edge_body(src_hbm, dst_hbm, asrc_hbm, adst_hbm, h_hbm,
               part_hbm, den_hbm,
               asrc_l, adst_l, srcb, dstb, wb, rows, zrow, zden,
               out_acc, den_acc, sem):
    cid = lax.axis_index("c")
    sid = lax.axis_index("s")
    wid = cid * NS + sid

    pltpu.sync_copy(asrc_hbm, asrc_l)
    pltpu.sync_copy(adst_hbm, adst_l)

    zero16 = jnp.zeros((16,), jnp.float32)

    def _zrow_body(i, _):
        for j in range(8):
            zrow[i, pl.ds(j * 16, 16)] = zero16
        return 0
    lax.fori_loop(0, ZR, _zrow_body, 0)

    def _zden_body(i, _):
        zden[pl.ds(i * 16, 16)] = zero16
        return 0
    lax.fori_loop(0, RPT // 16, _zden_body, 0)

    r0 = sid * RPT
    for t in range(RPT // ZR):
        pltpu.sync_copy(zrow, out_acc.at[pl.ds(r0 + t * ZR, ZR), :])
    pltpu.sync_copy(zden, den_acc.at[pl.ds(r0, RPT)])

    def _mx_body(i, m):
        ms = jnp.maximum(m[0], asrc_l[pl.ds(i * 16, 16)])
        md = jnp.maximum(m[1], adst_l[pl.ds(i * 16, 16)])
        return ms, md
    neg = jnp.full((16,), -3.0e38, jnp.float32)
    mx_s, mx_d = lax.fori_loop(0, NPAD // 16, _mx_body, (neg, neg))
    hs = mx_s[0]
    hd = mx_d[0]
    for l in range(1, 16):
        hs = jnp.maximum(hs, mx_s[l])
        hd = jnp.maximum(hd, mx_d[l])
    gmax = hs + hd
    c = jnp.where(gmax > 0, gmax, 0.2 * gmax)

    plsc.subcore_barrier()

    ebase = wid * EPW

    def _chunk_body(j, _):
        base = ebase + j * K
        pltpu.sync_copy(src_hbm.at[pl.ds(base, K)], srcb)
        pltpu.sync_copy(dst_hbm.at[pl.ds(base, K)], dstb)

        def _grp_body(g, _):
            si = srcb[pl.ds(g * 16, 16)]
            di = dstb[pl.ds(g * 16, 16)]
            av = plsc.load_gather(asrc_l, [si])
            bv = plsc.load_gather(adst_l, [di])
            gv = av + bv
            ev = jnp.where(gv > 0, gv, 0.2 * gv)
            eid = base + g * 16 + lax.iota(jnp.int32, 16)
            wv = jnp.where(eid < E, jnp.exp(ev - c), 0.0)
            wb[pl.ds(g * 16, 16)] = wv
            return 0
        lax.fori_loop(0, K // 16, _grp_body, 0)

        pltpu.async_copy(h_hbm.at[srcb], rows, sem).wait()

        def _scale_body(g, _):
            wv = wb[pl.ds(g * 16, 16)]
            for l in range(16):
                wvec = jnp.full((16,), wv[l], jnp.float32)
                r = g * 16 + l
                for j2 in range(8):
                    sl = pl.ds(j2 * 16, 16)
                    rows[r, sl] = rows[r, sl] * wvec
            return 0
        lax.fori_loop(0, K // 16, _scale_body, 0)

        pltpu.sync_copy(rows, out_acc.at[dstb], add=True)
        pltpu.sync_copy(wb, den_acc.at[dstb], add=True)
        return 0
    lax.fori_loop(0, NCHUNK, _chunk_body, 0)

    plsc.subcore_barrier()

    pltpu.sync_copy(out_acc.at[pl.ds(r0, RPT), :],
                    part_hbm.at[cid, pl.ds(r0, RPT), :])
    pltpu.sync_copy(den_acc.at[pl.ds(r0, RPT)],
                    den_hbm.at[cid, pl.ds(r0, RPT)])


_edge_pass = pl.kernel(
    _edge_body,
    out_type=(
        jax.ShapeDtypeStruct((NC, NPAD, D), jnp.float32),
        jax.ShapeDtypeStruct((NC, NPAD), jnp.float32),
    ),
    mesh=plsc.VectorSubcoreMesh(core_axis_name="c", subcore_axis_name="s",
                                num_cores=NC, num_subcores=NS),
    compiler_params=pltpu.CompilerParams(needs_layout_passes=False),
    scratch_types=[
        pltpu.VMEM((NPAD,), jnp.float32),
        pltpu.VMEM((NPAD,), jnp.float32),
        pltpu.VMEM((K,), jnp.int32),
        pltpu.VMEM((K,), jnp.int32),
        pltpu.VMEM((K,), jnp.float32),
        pltpu.VMEM((K, D), jnp.float32),
        pltpu.VMEM((ZR, D), jnp.float32),
        pltpu.VMEM((RPT,), jnp.float32),
        pltpu.VMEM_SHARED((NPAD, D), jnp.float32),
        pltpu.VMEM_SHARED((NPAD,), jnp.float32),
        pltpu.SemaphoreType.DMA,
    ],
)


@jax.jit
def kernel(edge_index, node_features, W1, att_src1, att_dst1, b1,
           W2, att_src2, att_dst2, b2):
    spread = (jnp.arange(EPAD - E, dtype=jnp.int32) * 97 + 13) % N
    src = jnp.concatenate([edge_index[0].astype(jnp.int32), spread])
    dst = jnp.concatenate([edge_index[1].astype(jnp.int32), spread])
    x = jnp.pad(node_features, ((0, NPAD - N), (0, 0)))

    h1, as1, ad1 = _project1(x, W1, att_src1, att_dst1)
    p1, d1 = _edge_pass(src, dst, as1, ad1, h1)
    h2, as2, ad2 = _project2(p1, d1, b1, W2, att_src2, att_dst2)
    p2, d2 = _edge_pass(src, dst, as2, ad2, h2)
    out = _finalize(p2, d2, b2)
    return out[:N]


# speedup vs baseline: 31.1515x; 1.0983x over previous
import functools

import jax
import jax.numpy as jnp
from jax import lax
from jax.experimental import pallas as pl
from jax.experimental.pallas import tpu as pltpu
from jax.experimental.pallas import tpu_sc as plsc

N = 10000
E = 320000
D = 128
NPAD = 10240
EPAD = 327680
NC = 2
NS = 16
NW = NC * NS
EPW = EPAD // NW
K = 64
NCHUNK = EPW // K
RPT = NPAD // NS
ZR = 64
BR = 512


def _proj_body(x_ref, w_ref, as_ref, ad_ref, h_ref, ao_ref, bo_ref):
    x = x_ref[...]
    h = lax.dot_general(x, w_ref[...], (((1,), (1,)), ((), ())),
                        preferred_element_type=jnp.float32)
    h_ref[...] = h
    ao_ref[...] = lax.dot_general(h, as_ref[...], (((1,), (0,)), ((), ())),
                                  preferred_element_type=jnp.float32)
    bo_ref[...] = lax.dot_general(h, ad_ref[...], (((1,), (0,)), ((), ())),
                                  preferred_element_type=jnp.float32)


def _project1(x, w, att_s, att_d):
    grid = NPAD // BR
    return pl.pallas_call(
        _proj_body,
        grid=(grid,),
        in_specs=[
            pl.BlockSpec((BR, D), lambda i: (i, 0)),
            pl.BlockSpec((D, D), lambda i: (0, 0)),
            pl.BlockSpec((D,), lambda i: (0,)),
            pl.BlockSpec((D,), lambda i: (0,)),
        ],
        out_specs=[
            pl.BlockSpec((BR, D), lambda i: (i, 0)),
            pl.BlockSpec((BR,), lambda i: (i,)),
            pl.BlockSpec((BR,), lambda i: (i,)),
        ],
        out_shape=[
            jax.ShapeDtypeStruct((NPAD, D), jnp.float32),
            jax.ShapeDtypeStruct((NPAD,), jnp.float32),
            jax.ShapeDtypeStruct((NPAD,), jnp.float32),
        ],
    )(x, w, att_s, att_d)


def _combine_proj_body(p_ref, d_ref, b_ref, w_ref, as_ref, ad_ref,
                       h_ref, ao_ref, bo_ref):
    ps = p_ref[0] + p_ref[1]
    den = jnp.maximum(d_ref[0] + d_ref[1], 1e-16)
    x = jnp.maximum(ps / den[:, None] + b_ref[...][None, :], 0.0)
    h = lax.dot_general(x, w_ref[...], (((1,), (1,)), ((), ())),
                        preferred_element_type=jnp.float32)
    h_ref[...] = h
    ao_ref[...] = lax.dot_general(h, as_ref[...], (((1,), (0,)), ((), ())),
                                  preferred_element_type=jnp.float32)
    bo_ref[...] = lax.dot_general(h, ad_ref[...], (((1,), (0,)), ((), ())),
                                  preferred_element_type=jnp.float32)


def _project2(part, den, b, w, att_s, att_d):
    grid = NPAD // BR
    return pl.pallas_call(
        _combine_proj_body,
        grid=(grid,),
        in_specs=[
            pl.BlockSpec((NC, BR, D), lambda i: (0, i, 0)),
            pl.BlockSpec((NC, BR), lambda i: (0, i)),
            pl.BlockSpec((D,), lambda i: (0,)),
            pl.BlockSpec((D, D), lambda i: (0, 0)),
            pl.BlockSpec((D,), lambda i: (0,)),
            pl.BlockSpec((D,), lambda i: (0,)),
        ],
        out_specs=[
            pl.BlockSpec((BR, D), lambda i: (i, 0)),
            pl.BlockSpec((BR,), lambda i: (i,)),
            pl.BlockSpec((BR,), lambda i: (i,)),
        ],
        out_shape=[
            jax.ShapeDtypeStruct((NPAD, D), jnp.float32),
            jax.ShapeDtypeStruct((NPAD,), jnp.float32),
            jax.ShapeDtypeStruct((NPAD,), jnp.float32),
        ],
    )(part, den, b, w, att_s, att_d)


def _final_body(p_ref, d_ref, b_ref, o_ref):
    ps = p_ref[0] + p_ref[1]
    den = jnp.maximum(d_ref[0] + d_ref[1], 1e-16)
    o_ref[...] = ps / den[:, None] + b_ref[...][None, :]


def _finalize(part, den, b):
    grid = NPAD // BR
    return pl.pallas_call(
        _final_body,
        grid=(grid,),
        in_specs=[
            pl.BlockSpec((NC, BR, D), lambda i: (0, i, 0)),
            pl.BlockSpec((NC, BR), lambda i: (0, i)),
            pl.BlockSpec((D,), lambda i: (0,)),
        ],
        out_specs=pl.BlockSpec((BR, D), lambda i: (i, 0)),
        out_shape=jax.ShapeDtypeStruct((NPAD, D), jnp.float32),
    )(part, den, b)


def _edge_body(src_hbm, dst_hbm, asrc_hbm, adst_hbm, h_hbm,
               part_hbm, den_hbm,
               asrc_l, adst_l, srcb, dstb, wb, rows, zden,
               out_acc, den_acc, sem):
    cid = lax.axis_index("c")
    sid = lax.axis_index("s")
    wid = cid * NS + sid

    pltpu.sync_copy(asrc_hbm, asrc_l)
    pltpu.sync_copy(adst_hbm, adst_l)

    zero16 = jnp.zeros((16,), jnp.float32)

    def _zrow_body(i, _):
        for j in range(8):
            rows[0, i, pl.ds(j * 16, 16)] = zero16
        return 0
    lax.fori_loop(0, K, _zrow_body, 0)

    def _zden_body(i, _):
        zden[pl.ds(i * 16, 16)] = zero16
        return 0
    lax.fori_loop(0, RPT // 16, _zden_body, 0)

    r0 = sid * RPT
    for t in range(RPT // K):
        pltpu.sync_copy(rows.at[0], out_acc.at[pl.ds(r0 + t * K, K), :])
    pltpu.sync_copy(zden, den_acc.at[pl.ds(r0, RPT)])

    def _mx_body(i, m):
        ms = jnp.maximum(m[0], asrc_l[pl.ds(i * 16, 16)])
        md = jnp.maximum(m[1], adst_l[pl.ds(i * 16, 16)])
        return ms, md
    neg = jnp.full((16,), -3.0e38, jnp.float32)
    mx_s, mx_d = lax.fori_loop(0, NPAD // 16, _mx_body, (neg, neg))
    hs = mx_s[0]
    hd = mx_d[0]
    for l in range(1, 16):
        hs = jnp.maximum(hs, mx_s[l])
        hd = jnp.maximum(hd, mx_d[l])
    gmax = hs + hd
    c = jnp.where(gmax > 0, gmax, 0.2 * gmax)

    plsc.subcore_barrier()

    ebase = wid * EPW

    def _copy_idx(j, b):
        base = ebase + j * K
        pltpu.sync_copy(src_hbm.at[pl.ds(base, K)], srcb.at[b])
        pltpu.sync_copy(dst_hbm.at[pl.ds(base, K)], dstb.at[b])

    def _start_gather(b):
        pltpu.async_copy(h_hbm.at[srcb.at[b]], rows.at[b], sem)

    _copy_idx(0, 0)
    _start_gather(0)

    def _pair_body(p, _):
        for b in range(2):
            j = 2 * p + b
            nb = 1 - b

            def _prefetch():
                _copy_idx(j + 1, nb)
                _start_gather(nb)
            if b == 0:
                _prefetch()
            else:
                pl.when(p < NCHUNK // 2 - 1)(_prefetch)

            base = ebase + j * K

            def _grp_body(g, _):
                si = srcb[b, pl.ds(g * 16, 16)]
                di = dstb[b, pl.ds(g * 16, 16)]
                av = plsc.load_gather(asrc_l, [si])
                bv = plsc.load_gather(adst_l, [di])
                gv = av + bv
                ev = jnp.where(gv > 0, gv, 0.2 * gv)
                eid = base + g * 16 + lax.iota(jnp.int32, 16)
                wv = jnp.where(eid < E, jnp.exp(ev - c), 0.0)
                wb[b, pl.ds(g * 16, 16)] = wv
                return 0
            lax.fori_loop(0, K // 16, _grp_body, 0)

            pltpu.make_async_copy(h_hbm.at[srcb.at[b]], rows.at[b], sem).wait()

            def _scale_body(g, _):
                wv = wb[b, pl.ds(g * 16, 16)]
                for l in range(16):
                    wvec = jnp.full((16,), wv[l], jnp.float32)
                    r = g * 16 + l
                    for j2 in range(8):
                        sl = pl.ds(j2 * 16, 16)
                        rows[b, r, sl] = rows[b, r, sl] * wvec
                return 0
            lax.fori_loop(0, K // 16, _scale_body, 0)

            pltpu.sync_copy(rows.at[b], out_acc.at[dstb.at[b]], add=True)
            pltpu.sync_copy(wb.at[b], den_acc.at[dstb.at[b]], add=True)
        return 0
    lax.fori_loop(0, NCHUNK // 2, _pair_body, 0)

    plsc.subcore_barrier()

    pltpu.sync_copy(out_acc.at[pl.ds(r0, RPT), :],
                    part_hbm.at[cid, pl.ds(r0, RPT), :])
    pltpu.sync_copy(den_acc.at[pl.ds(r0, RPT)],
                    den_hbm.at[cid, pl.ds(r0, RPT)])


_edge_pass = pl.kernel(
    _edge_body,
    out_type=(
        jax.ShapeDtypeStruct((NC, NPAD, D), jnp.float32),
        jax.ShapeDtypeStruct((NC, NPAD), jnp.float32),
    ),
    mesh=plsc.VectorSubcoreMesh(core_axis_name="c", subcore_axis_name="s",
                                num_cores=NC, num_subcores=NS),
    compiler_params=pltpu.CompilerParams(needs_layout_passes=False),
    scratch_types=[
        pltpu.VMEM((NPAD,), jnp.float32),
        pltpu.VMEM((NPAD,), jnp.float32),
        pltpu.VMEM((2, K), jnp.int32),
        pltpu.VMEM((2, K), jnp.int32),
        pltpu.VMEM((2, K), jnp.float32),
        pltpu.VMEM((2, K, D), jnp.float32),
        pltpu.VMEM((RPT,), jnp.float32),
        pltpu.VMEM_SHARED((NPAD, D), jnp.float32),
        pltpu.VMEM_SHARED((NPAD,), jnp.float32),
        pltpu.SemaphoreType.DMA,
    ],
)


@jax.jit
def kernel(edge_index, node_features, W1, att_src1, att_dst1, b1,
           W2, att_src2, att_dst2, b2):
    spread = (jnp.arange(EPAD - E, dtype=jnp.int32) * 97 + 13) % N
    src = jnp.concatenate([edge_index[0].astype(jnp.int32), spread])
    dst = jnp.concatenate([edge_index[1].astype(jnp.int32), spread])
    x = jnp.pad(node_features, ((0, NPAD - N), (0, 0)))

    h1, as1, ad1 = _project1(x, W1, att_src1, att_dst1)
    p1, d1 = _edge_pass(src, dst, as1, ad1, h1)
    h2, as2, ad2 = _project2(p1, d1, b1, W2, att_src2, att_dst2)
    p2, d2 = _edge_pass(src, dst, as2, ad2, h2)
    out = _finalize(p2, d2, b2)
    return out[:N]


# speedup vs baseline: 32.1022x; 1.0305x over previous
import functools

import jax
import jax.numpy as jnp
from jax import lax
from jax.experimental import pallas as pl
from jax.experimental.pallas import tpu as pltpu
from jax.experimental.pallas import tpu_sc as plsc

N = 10000
E = 320000
D = 128
NPAD = 10240
EPAD = 327680
NC = 2
NS = 16
NW = NC * NS
EPW = EPAD // NW
K = 64
NCHUNK = EPW // K
RPT = NPAD // NS
ZR = 64
BR = 512


def _proj_body(x_ref, w_ref, as_ref, ad_ref, h_ref, ao_ref, bo_ref):
    x = x_ref[...]
    h = lax.dot_general(x, w_ref[...], (((1,), (1,)), ((), ())),
                        preferred_element_type=jnp.float32)
    h_ref[...] = h
    ao_ref[...] = lax.dot_general(h, as_ref[...], (((1,), (0,)), ((), ())),
                                  preferred_element_type=jnp.float32)
    bo_ref[...] = lax.dot_general(h, ad_ref[...], (((1,), (0,)), ((), ())),
                                  preferred_element_type=jnp.float32)


def _project1(x, w, att_s, att_d):
    grid = NPAD // BR
    return pl.pallas_call(
        _proj_body,
        grid=(grid,),
        in_specs=[
            pl.BlockSpec((BR, D), lambda i: (i, 0)),
            pl.BlockSpec((D, D), lambda i: (0, 0)),
            pl.BlockSpec((D,), lambda i: (0,)),
            pl.BlockSpec((D,), lambda i: (0,)),
        ],
        out_specs=[
            pl.BlockSpec((BR, D), lambda i: (i, 0)),
            pl.BlockSpec((BR,), lambda i: (i,)),
            pl.BlockSpec((BR,), lambda i: (i,)),
        ],
        out_shape=[
            jax.ShapeDtypeStruct((NPAD, D), jnp.float32),
            jax.ShapeDtypeStruct((NPAD,), jnp.float32),
            jax.ShapeDtypeStruct((NPAD,), jnp.float32),
        ],
    )(x, w, att_s, att_d)


def _combine_proj_body(p_ref, d_ref, b_ref, w_ref, as_ref, ad_ref,
                       h_ref, ao_ref, bo_ref):
    ps = p_ref[0] + p_ref[1]
    den = jnp.maximum(d_ref[0] + d_ref[1], 1e-16)
    x = jnp.maximum(ps / den[:, None] + b_ref[...][None, :], 0.0)
    h = lax.dot_general(x, w_ref[...], (((1,), (1,)), ((), ())),
                        preferred_element_type=jnp.float32)
    h_ref[...] = h
    ao_ref[...] = lax.dot_general(h, as_ref[...], (((1,), (0,)), ((), ())),
                                  preferred_element_type=jnp.float32)
    bo_ref[...] = lax.dot_general(h, ad_ref[...], (((1,), (0,)), ((), ())),
                                  preferred_element_type=jnp.float32)


def _project2(part, den, b, w, att_s, att_d):
    grid = NPAD // BR
    return pl.pallas_call(
        _combine_proj_body,
        grid=(grid,),
        in_specs=[
            pl.BlockSpec((NC, BR, D), lambda i: (0, i, 0)),
            pl.BlockSpec((NC, BR), lambda i: (0, i)),
            pl.BlockSpec((D,), lambda i: (0,)),
            pl.BlockSpec((D, D), lambda i: (0, 0)),
            pl.BlockSpec((D,), lambda i: (0,)),
            pl.BlockSpec((D,), lambda i: (0,)),
        ],
        out_specs=[
            pl.BlockSpec((BR, D), lambda i: (i, 0)),
            pl.BlockSpec((BR,), lambda i: (i,)),
            pl.BlockSpec((BR,), lambda i: (i,)),
        ],
        out_shape=[
            jax.ShapeDtypeStruct((NPAD, D), jnp.float32),
            jax.ShapeDtypeStruct((NPAD,), jnp.float32),
            jax.ShapeDtypeStruct((NPAD,), jnp.float32),
        ],
    )(part, den, b, w, att_s, att_d)


def _final_body(p_ref, d_ref, b_ref, o_ref):
    ps = p_ref[0] + p_ref[1]
    den = jnp.maximum(d_ref[0] + d_ref[1], 1e-16)
    o_ref[...] = ps / den[:, None] + b_ref[...][None, :]


def _finalize(part, den, b):
    grid = NPAD // BR
    return pl.pallas_call(
        _final_body,
        grid=(grid,),
        in_specs=[
            pl.BlockSpec((NC, BR, D), lambda i: (0, i, 0)),
            pl.BlockSpec((NC, BR), lambda i: (0, i)),
            pl.BlockSpec((D,), lambda i: (0,)),
        ],
        out_specs=pl.BlockSpec((BR, D), lambda i: (i, 0)),
        out_shape=jax.ShapeDtypeStruct((NPAD, D), jnp.float32),
    )(part, den, b)


def _edge_body(src_hbm, dst_hbm, asrc_hbm, adst_hbm, h_hbm,
               part_hbm, den_hbm,
               asrc_l, adst_l, srcb, dstb, wb, rows, zden,
               out_acc, den_acc, sem, ssem):
    cid = lax.axis_index("c")
    sid = lax.axis_index("s")
    wid = cid * NS + sid

    pltpu.sync_copy(asrc_hbm, asrc_l)
    pltpu.sync_copy(adst_hbm, adst_l)

    zero16 = jnp.zeros((16,), jnp.float32)

    def _zrow_body(i, _):
        for j in range(8):
            rows[0, i, pl.ds(j * 16, 16)] = zero16
        return 0
    lax.fori_loop(0, K, _zrow_body, 0)

    def _zden_body(i, _):
        zden[pl.ds(i * 16, 16)] = zero16
        return 0
    lax.fori_loop(0, RPT // 16, _zden_body, 0)

    r0 = sid * RPT
    for t in range(RPT // K):
        pltpu.sync_copy(rows.at[0], out_acc.at[pl.ds(r0 + t * K, K), :])
    pltpu.sync_copy(zden, den_acc.at[pl.ds(r0, RPT)])

    def _mx_body(i, m):
        ms = jnp.maximum(m[0], asrc_l[pl.ds(i * 16, 16)])
        md = jnp.maximum(m[1], adst_l[pl.ds(i * 16, 16)])
        return ms, md
    neg = jnp.full((16,), -3.0e38, jnp.float32)
    mx_s, mx_d = lax.fori_loop(0, NPAD // 16, _mx_body, (neg, neg))
    hs = mx_s[0]
    hd = mx_d[0]
    for l in range(1, 16):
        hs = jnp.maximum(hs, mx_s[l])
        hd = jnp.maximum(hd, mx_d[l])
    gmax = hs + hd
    c = jnp.where(gmax > 0, gmax, 0.2 * gmax)

    plsc.subcore_barrier()

    ebase = wid * EPW

    def _copy_idx(j, b):
        base = ebase + j * K
        pltpu.sync_copy(src_hbm.at[pl.ds(base, K)], srcb.at[b])
        pltpu.sync_copy(dst_hbm.at[pl.ds(base, K)], dstb.at[b])

    def _start_gather(b):
        pltpu.async_copy(h_hbm.at[srcb.at[b]], rows.at[b], sem)

    _copy_idx(0, 0)
    _start_gather(0)

    def _wait_scatter(b):
        pltpu.make_async_copy(rows.at[b], out_acc.at[dstb.at[b]],
                              ssem).wait()
        pltpu.make_async_copy(wb.at[b], den_acc.at[dstb.at[b]],
                              ssem).wait()

    def _pair_body(p, _):
        for b in range(2):
            j = 2 * p + b
            nb = 1 - b

            def _prefetch():
                _copy_idx(j + 1, nb)
                _start_gather(nb)

            if b == 0:
                pl.when(p > 0)(lambda: _wait_scatter(nb))
                _prefetch()
            else:
                _wait_scatter(nb)
                pl.when(p < NCHUNK // 2 - 1)(_prefetch)

            base = ebase + j * K

            def _grp_body(g, _):
                si = srcb[b, pl.ds(g * 16, 16)]
                di = dstb[b, pl.ds(g * 16, 16)]
                av = plsc.load_gather(asrc_l, [si])
                bv = plsc.load_gather(adst_l, [di])
                gv = av + bv
                ev = jnp.where(gv > 0, gv, 0.2 * gv)
                eid = base + g * 16 + lax.iota(jnp.int32, 16)
                wv = jnp.where(eid < E, jnp.exp(ev - c), 0.0)
                wb[b, pl.ds(g * 16, 16)] = wv
                return 0
            lax.fori_loop(0, K // 16, _grp_body, 0)

            pltpu.make_async_copy(h_hbm.at[srcb.at[b]], rows.at[b], sem).wait()

            def _scale_body(g, _):
                wv = wb[b, pl.ds(g * 16, 16)]
                for l in range(16):
                    wvec = jnp.full((16,), wv[l], jnp.float32)
                    r = g * 16 + l
                    for j2 in range(8):
                        sl = pl.ds(j2 * 16, 16)
                        rows[b, r, sl] = rows[b, r, sl] * wvec
                return 0
            lax.fori_loop(0, K // 16, _scale_body, 0)

            pltpu.async_copy(rows.at[b], out_acc.at[dstb.at[b]], ssem,
                             add=True)
            pltpu.async_copy(wb.at[b], den_acc.at[dstb.at[b]], ssem,
                             add=True)
        return 0
    lax.fori_loop(0, NCHUNK // 2, _pair_body, 0)

    _wait_scatter(1)

    plsc.subcore_barrier()

    pltpu.sync_copy(out_acc.at[pl.ds(r0, RPT), :],
                    part_hbm.at[cid, pl.ds(r0, RPT), :])
    pltpu.sync_copy(den_acc.at[pl.ds(r0, RPT)],
                    den_hbm.at[cid, pl.ds(r0, RPT)])


_edge_pass = pl.kernel(
    _edge_body,
    out_type=(
        jax.ShapeDtypeStruct((NC, NPAD, D), jnp.float32),
        jax.ShapeDtypeStruct((NC, NPAD), jnp.float32),
    ),
    mesh=plsc.VectorSubcoreMesh(core_axis_name="c", subcore_axis_name="s",
                                num_cores=NC, num_subcores=NS),
    compiler_params=pltpu.CompilerParams(needs_layout_passes=False),
    scratch_types=[
        pltpu.VMEM((NPAD,), jnp.float32),
        pltpu.VMEM((NPAD,), jnp.float32),
        pltpu.VMEM((2, K), jnp.int32),
        pltpu.VMEM((2, K), jnp.int32),
        pltpu.VMEM((2, K), jnp.float32),
        pltpu.VMEM((2, K, D), jnp.float32),
        pltpu.VMEM((RPT,), jnp.float32),
        pltpu.VMEM_SHARED((NPAD, D), jnp.float32),
        pltpu.VMEM_SHARED((NPAD,), jnp.float32),
        pltpu.SemaphoreType.DMA,
        pltpu.SemaphoreType.DMA,
    ],
)


@jax.jit
def kernel(edge_index, node_features, W1, att_src1, att_dst1, b1,
           W2, att_src2, att_dst2, b2):
    spread = (jnp.arange(EPAD - E, dtype=jnp.int32) * 97 + 13) % N
    src = jnp.concatenate([edge_index[0].astype(jnp.int32), spread])
    dst = jnp.concatenate([edge_index[1].astype(jnp.int32), spread])
    x = jnp.pad(node_features, ((0, NPAD - N), (0, 0)))

    h1, as1, ad1 = _project1(x, W1, att_src1, att_dst1)
    p1, d1 = _edge_pass(src, dst, as1, ad1, h1)
    h2, as2, ad2 = _project2(p1, d1, b1, W2, att_src2, att_dst2)
    p2, d2 = _edge_pass(src, dst, as2, ad2, h2)
    out = _finalize(p2, d2, b2)
    return out[:N]


# speedup vs baseline: 55.3151x; 1.7231x over previous
import functools

import jax
import jax.numpy as jnp
from jax import lax
from jax.experimental import pallas as pl
from jax.experimental.pallas import tpu as pltpu
from jax.experimental.pallas import tpu_sc as plsc

N = 10000
E = 320000
D = 128
NPAD = 10240
EPAD = 327680
NC = 2
NS = 16
NW = NC * NS
EPW = EPAD // NW
K = 64
NCHUNK = EPW // K
SCH = 8
NSUP = NCHUNK // SCH
RPT = NPAD // NS
ZR = 64
BR = 512


def _proj_body(x_ref, w_ref, as_ref, ad_ref, h_ref, ao_ref, bo_ref):
    x = x_ref[...]
    h = lax.dot_general(x, w_ref[...], (((1,), (1,)), ((), ())),
                        preferred_element_type=jnp.float32)
    h_ref[...] = h
    ao_ref[...] = lax.dot_general(h, as_ref[...], (((1,), (0,)), ((), ())),
                                  preferred_element_type=jnp.float32)
    bo_ref[...] = lax.dot_general(h, ad_ref[...], (((1,), (0,)), ((), ())),
                                  preferred_element_type=jnp.float32)


def _project1(x, w, att_s, att_d):
    grid = NPAD // BR
    return pl.pallas_call(
        _proj_body,
        grid=(grid,),
        in_specs=[
            pl.BlockSpec((BR, D), lambda i: (i, 0)),
            pl.BlockSpec((D, D), lambda i: (0, 0)),
            pl.BlockSpec((D,), lambda i: (0,)),
            pl.BlockSpec((D,), lambda i: (0,)),
        ],
        out_specs=[
            pl.BlockSpec((BR, D), lambda i: (i, 0)),
            pl.BlockSpec((BR,), lambda i: (i,)),
            pl.BlockSpec((BR,), lambda i: (i,)),
        ],
        out_shape=[
            jax.ShapeDtypeStruct((NPAD, D), jnp.float32),
            jax.ShapeDtypeStruct((NPAD,), jnp.float32),
            jax.ShapeDtypeStruct((NPAD,), jnp.float32),
        ],
    )(x, w, att_s, att_d)


def _combine_proj_body(p_ref, d_ref, b_ref, w_ref, as_ref, ad_ref,
                       h_ref, ao_ref, bo_ref):
    ps = p_ref[0] + p_ref[1]
    den = jnp.maximum(d_ref[0] + d_ref[1], 1e-16)
    x = jnp.maximum(ps / den[:, None] + b_ref[...][None, :], 0.0)
    h = lax.dot_general(x, w_ref[...], (((1,), (1,)), ((), ())),
                        preferred_element_type=jnp.float32)
    h_ref[...] = h
    ao_ref[...] = lax.dot_general(h, as_ref[...], (((1,), (0,)), ((), ())),
                                  preferred_element_type=jnp.float32)
    bo_ref[...] = lax.dot_general(h, ad_ref[...], (((1,), (0,)), ((), ())),
                                  preferred_element_type=jnp.float32)


def _project2(part, den, b, w, att_s, att_d):
    grid = NPAD // BR
    return pl.pallas_call(
        _combine_proj_body,
        grid=(grid,),
        in_specs=[
            pl.BlockSpec((NC, BR, D), lambda i: (0, i, 0)),
            pl.BlockSpec((NC, BR), lambda i: (0, i)),
            pl.BlockSpec((D,), lambda i: (0,)),
            pl.BlockSpec((D, D), lambda i: (0, 0)),
            pl.BlockSpec((D,), lambda i: (0,)),
            pl.BlockSpec((D,), lambda i: (0,)),
        ],
        out_specs=[
            pl.BlockSpec((BR, D), lambda i: (i, 0)),
            pl.BlockSpec((BR,), lambda i: (i,)),
            pl.BlockSpec((BR,), lambda i: (i,)),
        ],
        out_shape=[
            jax.ShapeDtypeStruct((NPAD, D), jnp.float32),
            jax.ShapeDtypeStruct((NPAD,), jnp.float32),
            jax.ShapeDtypeStruct((NPAD,), jnp.float32),
        ],
    )(part, den, b, w, att_s, att_d)


def _final_body(p_ref, d_ref, b_ref, o_ref):
    ps = p_ref[0] + p_ref[1]
    den = jnp.maximum(d_ref[0] + d_ref[1], 1e-16)
    o_ref[...] = ps / den[:, None] + b_ref[...][None, :]


def _finalize(part, den, b):
    grid = NPAD // BR
    return pl.pallas_call(
        _final_body,
        grid=(grid,),
        in_specs=[
            pl.BlockSpec((NC, BR, D), lambda i: (0, i, 0)),
            pl.BlockSpec((NC, BR), lambda i: (0, i)),
            pl.BlockSpec((D,), lambda i: (0,)),
        ],
        out_specs=pl.BlockSpec((BR, D), lambda i: (i, 0)),
        out_shape=jax.ShapeDtypeStruct((NPAD, D), jnp.float32),
    )(part, den, b)


def _edge_body(src_hbm, dst_hbm, asrc_hbm, adst_hbm, h_hbm,
               part_hbm, den_hbm,
               asrc_l, adst_l, sidx, wb, rows, zden,
               out_acc, den_acc, sem, ssem, isem):
    cid = lax.axis_index("c")
    sid = lax.axis_index("s")
    wid = cid * NS + sid

    pltpu.sync_copy(asrc_hbm, asrc_l)
    pltpu.sync_copy(adst_hbm, adst_l)

    zero16 = jnp.zeros((16,), jnp.float32)

    def _zrow_body(i, _):
        for j in range(8):
            rows[0, i, pl.ds(j * 16, 16)] = zero16
        return 0
    lax.fori_loop(0, K, _zrow_body, 0)

    def _zden_body(i, _):
        zden[pl.ds(i * 16, 16)] = zero16
        return 0
    lax.fori_loop(0, RPT // 16, _zden_body, 0)

    r0 = sid * RPT
    for t in range(RPT // K):
        pltpu.sync_copy(rows.at[0], out_acc.at[pl.ds(r0 + t * K, K), :])
    pltpu.sync_copy(zden, den_acc.at[pl.ds(r0, RPT)])

    def _mx_body(i, m):
        ms = jnp.maximum(m[0], asrc_l[pl.ds(i * 16, 16)])
        md = jnp.maximum(m[1], adst_l[pl.ds(i * 16, 16)])
        return ms, md
    neg = jnp.full((16,), -3.0e38, jnp.float32)
    mx_s, mx_d = lax.fori_loop(0, NPAD // 16, _mx_body, (neg, neg))
    hs = mx_s[0]
    hd = mx_d[0]
    for l in range(1, 16):
        hs = jnp.maximum(hs, mx_s[l])
        hd = jnp.maximum(hd, mx_d[l])
    gmax = hs + hd
    c = jnp.where(gmax > 0, gmax, 0.2 * gmax)

    plsc.subcore_barrier()

    ebase = wid * EPW

    def _copy_sidx(s, slot):
        pltpu.async_copy(src_hbm.at[wid, s], sidx.at[slot, 0], isem)
        pltpu.async_copy(dst_hbm.at[wid, s], sidx.at[slot, 1], isem)

    def _wait_sidx(s, slot):
        pltpu.make_async_copy(src_hbm.at[wid, s], sidx.at[slot, 0],
                              isem).wait()
        pltpu.make_async_copy(dst_hbm.at[wid, s], sidx.at[slot, 1],
                              isem).wait()

    def _start_gather(slot, cc, b):
        pltpu.async_copy(h_hbm.at[sidx.at[slot, 0, cc]], rows.at[b], sem)

    def _wait_gather(slot, cc, b):
        pltpu.make_async_copy(h_hbm.at[sidx.at[slot, 0, cc]], rows.at[b],
                              sem).wait()

    def _wait_scatter(b):
        pltpu.make_async_copy(rows.at[b], out_acc.at[sidx.at[0, 1, 0]],
                              ssem).wait()
        pltpu.make_async_copy(wb.at[b], den_acc.at[sidx.at[0, 1, 0]],
                              ssem).wait()

    _copy_sidx(0, 0)
    _wait_sidx(0, 0)
    _start_gather(0, 0, 0)

    def _sup_body(s, _):
        slot = s % 2
        nslot = 1 - slot
        for cc in range(SCH):
            j8 = s * SCH + cc
            b = cc % 2
            nb = 1 - b

            if cc < 2:
                pl.when(s > 0)(lambda bb=b: _wait_scatter(bb))
            else:
                _wait_scatter(b)

            if cc == 2:
                pl.when(s < NSUP - 1)(lambda: _copy_sidx(s + 1, nslot))

            if cc < SCH - 1:
                _start_gather(slot, cc + 1, nb)
            else:
                def _next_sup():
                    _wait_sidx(s + 1, nslot)
                    _start_gather(nslot, 0, nb)
                pl.when(s < NSUP - 1)(_next_sup)

            base = ebase + j8 * K

            def _grp_body(g, _):
                si = sidx[slot, 0, cc, pl.ds(g * 16, 16)]
                di = sidx[slot, 1, cc, pl.ds(g * 16, 16)]
                av = plsc.load_gather(asrc_l, [si])
                bv = plsc.load_gather(adst_l, [di])
                gv = av + bv
                ev = jnp.where(gv > 0, gv, 0.2 * gv)
                eid = base + g * 16 + lax.iota(jnp.int32, 16)
                wv = jnp.where(eid < E, jnp.exp(ev - c), 0.0)
                wb[b, pl.ds(g * 16, 16)] = wv
                return 0
            lax.fori_loop(0, K // 16, _grp_body, 0)

            _wait_gather(slot, cc, b)

            def _scale_body(g, _):
                wv = wb[b, pl.ds(g * 16, 16)]
                for l in range(16):
                    wvec = jnp.full((16,), wv[l], jnp.float32)
                    r = g * 16 + l
                    for j2 in range(8):
                        sl = pl.ds(j2 * 16, 16)
                        rows[b, r, sl] = rows[b, r, sl] * wvec
                return 0
            lax.fori_loop(0, K // 16, _scale_body, 0)

            pltpu.async_copy(rows.at[b], out_acc.at[sidx.at[slot, 1, cc]],
                             ssem, add=True)
            pltpu.async_copy(wb.at[b], den_acc.at[sidx.at[slot, 1, cc]],
                             ssem, add=True)
        return 0
    lax.fori_loop(0, NSUP, _sup_body, 0)

    _wait_scatter(0)
    _wait_scatter(1)

    plsc.subcore_barrier()

    pltpu.sync_copy(out_acc.at[pl.ds(r0, RPT), :],
                    part_hbm.at[cid, pl.ds(r0, RPT), :])
    pltpu.sync_copy(den_acc.at[pl.ds(r0, RPT)],
                    den_hbm.at[cid, pl.ds(r0, RPT)])


_edge_pass = pl.kernel(
    _edge_body,
    out_type=(
        jax.ShapeDtypeStruct((NC, NPAD, D), jnp.float32),
        jax.ShapeDtypeStruct((NC, NPAD), jnp.float32),
    ),
    mesh=plsc.VectorSubcoreMesh(core_axis_name="c", subcore_axis_name="s",
                                num_cores=NC, num_subcores=NS),
    compiler_params=pltpu.CompilerParams(needs_layout_passes=False),
    scratch_types=[
        pltpu.VMEM((NPAD,), jnp.float32),
        pltpu.VMEM((NPAD,), jnp.float32),
        pltpu.VMEM((2, 2, SCH, K), jnp.int32),
        pltpu.VMEM((2, K), jnp.float32),
        pltpu.VMEM((2, K, D), jnp.float32),
        pltpu.VMEM((RPT,), jnp.float32),
        pltpu.VMEM_SHARED((NPAD, D), jnp.float32),
        pltpu.VMEM_SHARED((NPAD,), jnp.float32),
        pltpu.SemaphoreType.DMA,
        pltpu.SemaphoreType.DMA,
        pltpu.SemaphoreType.DMA,
    ],
)


@jax.jit
def kernel(edge_index, node_features, W1, att_src1, att_dst1, b1,
           W2, att_src2, att_dst2, b2):
    spread = (jnp.arange(EPAD - E, dtype=jnp.int32) * 97 + 13) % N
    src = jnp.concatenate([edge_index[0].astype(jnp.int32), spread])
    dst = jnp.concatenate([edge_index[1].astype(jnp.int32), spread])
    src = src.reshape(NW, NSUP, SCH, K)
    dst = dst.reshape(NW, NSUP, SCH, K)
    x = jnp.pad(node_features, ((0, NPAD - N), (0, 0)))

    h1, as1, ad1 = _project1(x, W1, att_src1, att_dst1)
    p1, d1 = _edge_pass(src, dst, as1, ad1, h1)
    h2, as2, ad2 = _project2(p1, d1, b1, W2, att_src2, att_dst2)
    p2, d2 = _edge_pass(src, dst, as2, ad2, h2)
    out = _finalize(p2, d2, b2)
    return out[:N]
